# baseline (speedup 1.0000x reference)
import functools

import jax
import jax.numpy as jnp
from jax import lax
from jax.experimental import pallas as pl
from jax.experimental.pallas import tpu as pltpu
from jax.experimental.pallas import tpu_sc as plsc

N = 10000
E = 320000
D = 128
G = 64

NC = 2
NS = 16
K = 125
EPT = E // (NC * NS)
CH = EPT // K
GR = 8
NG = CH // GR
NPAD = 10240
RPT = NPAD // NS


def _make_agg():
    mesh = plsc.VectorSubcoreMesh(core_axis_name="c", subcore_axis_name="s")

    @functools.partial(
        pl.kernel,
        out_type=jax.ShapeDtypeStruct((NC * NPAD, D), jnp.float32),
        mesh=mesh,
        scratch_types=[
            pltpu.VMEM((CH, K), jnp.int32),
            pltpu.VMEM((GR, K), jnp.int32),
            pltpu.VMEM((GR, K), jnp.int32),
            pltpu.VMEM((K, D), jnp.float32),
            pltpu.VMEM((K, D), jnp.float32),
            pltpu.VMEM_SHARED((NPAD, D), jnp.float32),
            pltpu.SemaphoreType.DMA,
            pltpu.SemaphoreType.DMA,
            pltpu.SemaphoreType.DMA,
            pltpu.SemaphoreType.DMA,
            pltpu.SemaphoreType.DMA,
            pltpu.SemaphoreType.DMA,
            pltpu.SemaphoreType.DMA,
            pltpu.SemaphoreType.DMA,
        ],
    )
    def agg(x_hbm, e_hbm, zero_hbm, out_hbm,
            src_v, ring0, ring1, rows0, rows1, acc_sh,
            gsa0, gsa1, gsb0, gsb1, dsem0, dsem1, zsem, stsem):
        c = lax.axis_index("c")
        s = lax.axis_index("s")
        row0 = c * (NS * CH) + s * CH
        rows = (rows0, rows1)
        gsemsa = (gsa0, gsa1)
        gsemsb = (gsb0, gsb1)
        rings = (ring0, ring1)
        dsems = (dsem0, dsem1)
        zcopy = pltpu.make_async_copy(zero_hbm, acc_sh.at[pl.ds(s * RPT, RPT)],
                                      zsem)
        zcopy.start()
        stage = pltpu.make_async_copy(e_hbm.at[0, pl.ds(row0, CH)], src_v,
                                      stsem)
        stage.start()

        KA = 64

        def gather(j, b):
            ca = pltpu.make_async_copy(
                x_hbm.at[src_v.at[j, pl.ds(0, KA)]],
                rows[b].at[pl.ds(0, KA)], gsemsa[b])
            cb = pltpu.make_async_copy(
                x_hbm.at[src_v.at[j, pl.ds(KA, K - KA)]],
                rows[b].at[pl.ds(KA, K - KA)], gsemsb[b])

            class _Pair:
                def start(self):
                    ca.start()
                    cb.start()

                def wait(self):
                    ca.wait()
                    cb.wait()

            return _Pair()

        def dfetch(g, gs):
            return pltpu.make_async_copy(
                e_hbm.at[1, pl.ds(row0 + g * GR, GR)], rings[gs], dsems[gs])

        dfetch(0, 0).start()
        dfetch(1, 1).start()
        stage.wait()
        for b in range(2):
            gather(b, b).start()
        zcopy.wait()
        plsc.subcore_barrier()

        def chunk_ops(g, gs, b8, last_pair):
            j = g * GR + b8
            b = b8 % 2
            gather(j, b).wait()
            pltpu.sync_copy(rows[b], acc_sh.at[rings[gs].at[b8]], add=True)
            if not (last_pair and g == NG - 1 and b8 >= GR - 2):
                gather(j + 2, b).start()

        def group_ops(g, gs, last_pair):
            dfetch(g, gs).wait()
            for b8 in range(GR):
                chunk_ops(g, gs, b8, last_pair)
            if not last_pair:
                dfetch(g + 2, gs).start()

        for gs in range(2):
            group_ops(gs, gs, False)

        def body(g2, carry):
            for gs in range(2):
                group_ops(g2 * 2 + gs, gs, False)
            return carry

        lax.fori_loop(1, NG // 2 - 1, body, 0)
        for gs in range(2):
            group_ops(NG - 2 + gs, gs, True)
        plsc.subcore_barrier()
        pltpu.sync_copy(acc_sh.at[pl.ds(s * RPT, RPT)],
                        out_hbm.at[pl.ds(c * NPAD + s * RPT, RPT)])

    return agg


_agg = _make_agg()


def _dense_body(x_ref, a_ref, w_ref, b_ref, g_ref, be_ref, o_ref):
    xs = (x_ref[...] +
          a_ref[pl.ds(0, N), :] +
          a_ref[pl.ds(NPAD, N), :])
    h = lax.dot_general(xs, w_ref[...], (((1,), (1,)), ((), ())),
                        preferred_element_type=jnp.float32) + b_ref[...]
    mu = jnp.mean(h, axis=0, keepdims=True)
    var = jnp.mean((h - mu) ** 2, axis=0, keepdims=True)
    hn = g_ref[...] * (h - mu) * lax.rsqrt(var + 1e-5) + be_ref[...]
    o_ref[...] = jnp.maximum(hn, 0.0)


_dense = pl.pallas_call(
    _dense_body, out_shape=jax.ShapeDtypeStruct((N, D), jnp.float32))


def _final_body(x_ref, a_ref, w_ref, b_ref, g_ref, be_ref,
                batch_ref, wf_ref, bf_ref, o_ref):
    xs = (x_ref[...] +
          a_ref[pl.ds(0, N), :] +
          a_ref[pl.ds(NPAD, N), :])
    h = lax.dot_general(xs, w_ref[...], (((1,), (1,)), ((), ())),
                        preferred_element_type=jnp.float32) + b_ref[...]
    mu = jnp.mean(h, axis=0, keepdims=True)
    var = jnp.mean((h - mu) ** 2, axis=0, keepdims=True)
    hn = g_ref[...] * (h - mu) * lax.rsqrt(var + 1e-5) + be_ref[...]
    h = jnp.maximum(hn, 0.0)
    onehot = (batch_ref[...] == lax.broadcasted_iota(jnp.int32, (G, N), 0)
              ).astype(jnp.float32)
    sums = lax.dot_general(onehot, h, (((1,), (0,)), ((), ())),
                           preferred_element_type=jnp.float32)
    counts = jnp.sum(onehot, axis=1, keepdims=True)
    pooled = sums / jnp.maximum(counts, 1.0)
    o_ref[...] = lax.dot_general(pooled, wf_ref[...], (((1,), (1,)), ((), ())),
                                 preferred_element_type=jnp.float32) + bf_ref[...]


_final = pl.pallas_call(
    _final_body, out_shape=jax.ShapeDtypeStruct((G, D), jnp.float32))


def kernel(x, edge_index, batch, W1, b1, g1, be1, W2, b2, g2, be2,
           W3, b3, g3, be3, Wf, bf):
    e3 = edge_index.reshape(2, E // K, K)
    zero = jnp.zeros((RPT, D), jnp.float32)
    batch2 = batch.reshape(1, N)
    b1r, g1r, be1r = b1.reshape(1, D), g1.reshape(1, D), be1.reshape(1, D)
    b2r, g2r, be2r = b2.reshape(1, D), g2.reshape(1, D), be2.reshape(1, D)
    b3r, g3r, be3r = b3.reshape(1, D), g3.reshape(1, D), be3.reshape(1, D)
    bfr = bf.reshape(1, D)

    a = _agg(x, e3, zero)
    h = _dense(x, a, W1, b1r, g1r, be1r)
    a = _agg(h, e3, zero)
    h = _dense(h, a, W2, b2r, g2r, be2r)
    a = _agg(h, e3, zero)
    return _final(h, a, W3, b3r, g3r, be3r, batch2, Wf, bfr)


# speedup vs baseline: 1.0982x; 1.0982x over previous
import functools

import jax
import jax.numpy as jnp
from jax import lax
from jax.experimental import pallas as pl
from jax.experimental.pallas import tpu as pltpu
from jax.experimental.pallas import tpu_sc as plsc

N = 10000
E = 320000
D = 128
G = 64

NC = 2
NS = 16
K = 50
EPT = E // (NC * NS)
CH = EPT // K
GR = 8
NG = CH // GR
NB = 4
NPAD = 10240
RPT = NPAD // NS


def _make_agg():
    mesh = plsc.VectorSubcoreMesh(core_axis_name="c", subcore_axis_name="s")

    @functools.partial(
        pl.kernel,
        out_type=jax.ShapeDtypeStruct((NC * NPAD, D), jnp.float32),
        mesh=mesh,
        scratch_types=[
            pltpu.VMEM((GR, K), jnp.int32),
            pltpu.VMEM((GR, K), jnp.int32),
            pltpu.VMEM((GR, K), jnp.int32),
            pltpu.VMEM((GR, K), jnp.int32),
            pltpu.VMEM((K, D), jnp.float32),
            pltpu.VMEM((K, D), jnp.float32),
            pltpu.VMEM((K, D), jnp.float32),
            pltpu.VMEM((K, D), jnp.float32),
            pltpu.VMEM_SHARED((NPAD, D), jnp.float32),
            pltpu.SemaphoreType.DMA,
            pltpu.SemaphoreType.DMA,
            pltpu.SemaphoreType.DMA,
            pltpu.SemaphoreType.DMA,
            pltpu.SemaphoreType.DMA,
            pltpu.SemaphoreType.DMA,
            pltpu.SemaphoreType.DMA,
            pltpu.SemaphoreType.DMA,
            pltpu.SemaphoreType.DMA,
        ],
    )
    def agg(x_hbm, e_hbm, zero_hbm, out_hbm,
            sring0, sring1, dring0, dring1, rows0, rows1, rows2, rows3,
            acc_sh, gsem0, gsem1, gsem2, gsem3, ssem0, ssem1, dsem0, dsem1,
            zsem):
        c = lax.axis_index("c")
        s = lax.axis_index("s")
        row0 = c * (NS * CH) + s * CH
        rows = (rows0, rows1, rows2, rows3)
        gsems = (gsem0, gsem1, gsem2, gsem3)
        srings = (sring0, sring1)
        ssems = (ssem0, ssem1)
        drings = (dring0, dring1)
        dsems = (dsem0, dsem1)
        zcopy = pltpu.make_async_copy(zero_hbm, acc_sh.at[pl.ds(s * RPT, RPT)],
                                      zsem)
        zcopy.start()

        def sfetch(g, gs):
            return pltpu.make_async_copy(
                e_hbm.at[0, pl.ds(row0 + g * GR, GR)], srings[gs], ssems[gs])

        def dfetch(g, gs):
            return pltpu.make_async_copy(
                e_hbm.at[1, pl.ds(row0 + g * GR, GR)], drings[gs], dsems[gs])

        def gather(g, gs, b8, b):
            return pltpu.make_async_copy(x_hbm.at[srings[gs].at[b8]],
                                         rows[b], gsems[b])

        sfetch(0, 0).start()
        dfetch(0, 0).start()
        dfetch(1, 1).start()
        sfetch(0, 0).wait()
        for b in range(NB):
            gather(0, 0, b, b).start()
        zcopy.wait()
        plsc.subcore_barrier()

        def chunk_ops(g, gs, b8, last_group):
            b = b8 % NB
            gather(g, gs, b8, b).wait()
            pltpu.sync_copy(rows[b], acc_sh.at[drings[gs].at[b8]], add=True)
            if b8 + NB < GR:
                gather(g, gs, b8 + NB, b).start()
            elif not last_group:
                gather(g + 1, 1 - gs, b8 + NB - GR, b).start()

        def group_ops(g, gs, tail_kind):
            if tail_kind != 2:
                sfetch(g + 1, 1 - gs).start()
            dfetch(g, gs).wait()
            for b8 in range(GR):
                if b8 == NB and tail_kind != 2:
                    sfetch(g + 1, 1 - gs).wait()
                chunk_ops(g, gs, b8, tail_kind == 2)
            if tail_kind == 0:
                dfetch(g + 2, gs).start()

        for gs in range(2):
            group_ops(gs, gs, 0)

        def body(g2, carry):
            for gs in range(2):
                group_ops(g2 * 2 + gs, gs, 0)
            return carry

        lax.fori_loop(1, (NG - 3) // 2, body, 0)
        group_ops(NG - 3, 0, 0)
        group_ops(NG - 2, 1, 1)
        group_ops(NG - 1, 0, 2)
        plsc.subcore_barrier()
        pltpu.sync_copy(acc_sh.at[pl.ds(s * RPT, RPT)],
                        out_hbm.at[pl.ds(c * NPAD + s * RPT, RPT)])

    return agg


_agg = _make_agg()


def _layer(x_ref, a_ref, w_ref, b_ref, g_ref, be_ref):
    xs = (x_ref[...] +
          a_ref[pl.ds(0, N), :] +
          a_ref[pl.ds(NPAD, N), :])
    h = lax.dot_general(xs, w_ref[...], (((1,), (1,)), ((), ())),
                        preferred_element_type=jnp.float32) + b_ref[...]
    mu = jnp.mean(h, axis=0, keepdims=True)
    var = jnp.mean((h - mu) ** 2, axis=0, keepdims=True)
    hn = g_ref[...] * (h - mu) * lax.rsqrt(var + 1e-5) + be_ref[...]
    return jnp.maximum(hn, 0.0)


def _dense_body(x_ref, a_ref, w_ref, b_ref, g_ref, be_ref, o_ref):
    o_ref[...] = _layer(x_ref, a_ref, w_ref, b_ref, g_ref, be_ref)


_dense = pl.pallas_call(
    _dense_body, out_shape=jax.ShapeDtypeStruct((N, D), jnp.float32))


def _final_body(x_ref, a_ref, w_ref, b_ref, g_ref, be_ref,
                batch_ref, wf_ref, bf_ref, o_ref):
    h = _layer(x_ref, a_ref, w_ref, b_ref, g_ref, be_ref)
    onehot = (batch_ref[...] == lax.broadcasted_iota(jnp.int32, (G, N), 0)
              ).astype(jnp.float32)
    sums = lax.dot_general(onehot, h, (((1,), (0,)), ((), ())),
                           preferred_element_type=jnp.float32)
    counts = jnp.sum(onehot, axis=1, keepdims=True)
    pooled = sums / jnp.maximum(counts, 1.0)
    o_ref[...] = lax.dot_general(pooled, wf_ref[...], (((1,), (1,)), ((), ())),
                                 preferred_element_type=jnp.float32) + bf_ref[...]


_final = pl.pallas_call(
    _final_body, out_shape=jax.ShapeDtypeStruct((G, D), jnp.float32))


def kernel(x, edge_index, batch, W1, b1, g1, be1, W2, b2, g2, be2,
           W3, b3, g3, be3, Wf, bf):
    e3 = edge_index.reshape(2, E // K, K)
    zero = jnp.zeros((RPT, D), jnp.float32)
    batch2 = batch.reshape(1, N)
    b1r, g1r, be1r = b1.reshape(1, D), g1.reshape(1, D), be1.reshape(1, D)
    b2r, g2r, be2r = b2.reshape(1, D), g2.reshape(1, D), be2.reshape(1, D)
    b3r, g3r, be3r = b3.reshape(1, D), g3.reshape(1, D), be3.reshape(1, D)
    bfr = bf.reshape(1, D)

    a = _agg(x, e3, zero)
    h = _dense(x, a, W1, b1r, g1r, be1r)
    a = _agg(h, e3, zero)
    h = _dense(h, a, W2, b2r, g2r, be2r)
    a = _agg(h, e3, zero)
    return _final(h, a, W3, b3r, g3r, be3r, batch2, Wf, bfr)
